# baseline (speedup 1.0000x reference)
import functools

import jax
import jax.numpy as jnp
from jax import lax
from jax.experimental import pallas as pl
from jax.experimental.pallas import tpu as pltpu
from jax.experimental.pallas import tpu_sc as plsc

N_NODES = 10000
D = 128
B = 320000
NC = 2
NS = 16
NW = NC * NS
B_PER_W = B // NW
C = 80
N_CHUNKS = B_PER_W // C
G = C // 16
UNROLL = 8
NACC = 4


def _compute_chunk(rows_s, rows_d, out_v, out_base):
    lane = lax.iota(jnp.int32, 16)

    def g_body(g, _):
        def q_body(q, tot):
            for jj in range(4):
                j = q * 4 + jj
                e = g * 16 + j
                parts = [
                    rows_s[e, pl.ds(k * 16, 16)] * rows_d[e, pl.ds(k * 16, 16)]
                    for k in range(D // 16)
                ]
                v = ((parts[0] + parts[1]) + (parts[2] + parts[3])) + (
                    (parts[4] + parts[5]) + (parts[6] + parts[7])
                )
                red = jnp.sum(v)
                tot = jnp.where(lane == j, red, tot)
            return tot

        tot = lax.fori_loop(0, 4, q_body, jnp.zeros((16,), jnp.float32))
        out_v[pl.ds(out_base + g * 16, 16)] = tot
        return 0

    lax.fori_loop(0, G, g_body, 0)


def _make_sc_kernel():
    mesh = plsc.VectorSubcoreMesh(core_axis_name="c", subcore_axis_name="s")

    @functools.partial(
        pl.kernel,
        mesh=mesh,
        compiler_params=pltpu.CompilerParams(needs_layout_passes=False),
        out_type=jax.ShapeDtypeStruct((B,), jnp.float32),
        scratch_types=[
            pltpu.VMEM((B_PER_W,), jnp.int32),
            pltpu.VMEM((B_PER_W,), jnp.int32),
            pltpu.VMEM((C, D), jnp.float32),
            pltpu.VMEM((C, D), jnp.float32),
            pltpu.VMEM((C, D), jnp.float32),
            pltpu.VMEM((C, D), jnp.float32),
            pltpu.VMEM((B_PER_W,), jnp.float32),
            pltpu.SemaphoreType.DMA,
            pltpu.SemaphoreType.DMA,
            pltpu.SemaphoreType.DMA,
            pltpu.SemaphoreType.DMA,
        ],
    )
    def k(x_hbm, src_hbm, dst_hbm, out_hbm,
          idx_s, idx_d, rs0, rs1, rd0, rd1, out_v,
          sem_s0, sem_s1, sem_d0, sem_d1):
        wid = lax.axis_index("s") * NC + lax.axis_index("c")
        wbase = wid * B_PER_W

        pltpu.sync_copy(src_hbm.at[pl.ds(wbase, B_PER_W)], idx_s)
        pltpu.sync_copy(dst_hbm.at[pl.ds(wbase, B_PER_W)], idx_d)

        bufs = ((rs0, rd0, sem_s0, sem_d0), (rs1, rd1, sem_s1, sem_d1))

        def issue(c, buf):
            rs, rd, ss, sd = buf
            cbase = pl.multiple_of(c * C, C)
            pltpu.async_copy(x_hbm.at[idx_s.at[pl.ds(cbase, C)]], rs, ss)
            pltpu.async_copy(x_hbm.at[idx_d.at[pl.ds(cbase, C)]], rd, sd)

        def wait_and_compute(c, buf):
            rs, rd, ss, sd = buf
            pltpu.make_async_copy(x_hbm.at[idx_s.at[pl.ds(0, C)]], rs, ss).wait()
            pltpu.make_async_copy(x_hbm.at[idx_d.at[pl.ds(0, C)]], rd, sd).wait()
            _compute_chunk(rs, rd, out_v, c * C)

        issue(0, bufs[0])

        def pair_body(i, _):
            c = i * 2
            issue(c + 1, bufs[1])
            wait_and_compute(c, bufs[0])
            issue(c + 2, bufs[0])
            wait_and_compute(c + 1, bufs[1])
            return 0

        lax.fori_loop(0, (N_CHUNKS - 1) // 2, pair_body, 0)
        wait_and_compute(N_CHUNKS - 1, bufs[0])

        pltpu.sync_copy(out_v, out_hbm.at[pl.ds(wbase, B_PER_W)])

    return k


_sc_kernel = _make_sc_kernel()


@jax.jit
def kernel(x, edge_index):
    ei = edge_index.astype(jnp.int32)
    return _sc_kernel(x, ei[0], ei[1])


# speedup vs baseline: 1.1372x; 1.1372x over previous
import functools

import jax
import jax.numpy as jnp
from jax import lax
from jax.experimental import pallas as pl
from jax.experimental.pallas import tpu as pltpu
from jax.experimental.pallas import tpu_sc as plsc

N_NODES = 10000
D = 128
B = 320000
NC = 2
NS = 16
NW = NC * NS
B_PER_W = B // NW
C = 80
N_CHUNKS = B_PER_W // C
G = C // 16
UNROLL = 8
NACC = 4


def _compute_chunk(rows_s, rows_d, out_v, out_base):
    lane = lax.iota(jnp.int32, 16)

    def g_body(g, _):
        def q_body(q, tot):
            for jj in range(4):
                j = q * 4 + jj
                e = g * 16 + j
                acc0 = jnp.zeros((16,), jnp.float32)
                acc1 = jnp.zeros((16,), jnp.float32)
                for k in range(D // 32):
                    s = plsc.bitcast(rows_s[e, pl.ds(k * 16, 16)], jnp.bfloat16)
                    t = plsc.bitcast(rows_d[e, pl.ds(k * 16, 16)], jnp.bfloat16)
                    a, b = plsc.unpack(s * t, format=plsc.PackFormat.INTERLEAVED)
                    acc0 = acc0 + a
                    acc1 = acc1 + b
                red = jnp.sum(acc0 + acc1)
                tot = jnp.where(lane == j, red, tot)
            return tot

        tot = lax.fori_loop(0, 4, q_body, jnp.zeros((16,), jnp.float32))
        out_v[pl.ds(out_base + g * 16, 16)] = tot
        return 0

    lax.fori_loop(0, G, g_body, 0)


def _make_sc_kernel():
    mesh = plsc.VectorSubcoreMesh(core_axis_name="c", subcore_axis_name="s")

    @functools.partial(
        pl.kernel,
        mesh=mesh,
        compiler_params=pltpu.CompilerParams(
            needs_layout_passes=False, use_tc_tiling_on_sc=False
        ),
        out_type=jax.ShapeDtypeStruct((B,), jnp.float32),
        scratch_types=[
            pltpu.VMEM((B_PER_W,), jnp.int32),
            pltpu.VMEM((B_PER_W,), jnp.int32),
            pltpu.VMEM((C, D // 2), jnp.int32),
            pltpu.VMEM((C, D // 2), jnp.int32),
            pltpu.VMEM((C, D // 2), jnp.int32),
            pltpu.VMEM((C, D // 2), jnp.int32),
            pltpu.VMEM((B_PER_W,), jnp.float32),
            pltpu.SemaphoreType.DMA,
            pltpu.SemaphoreType.DMA,
            pltpu.SemaphoreType.DMA,
            pltpu.SemaphoreType.DMA,
        ],
    )
    def k(x_hbm, src_hbm, dst_hbm, out_hbm,
          idx_s, idx_d, rs0, rs1, rd0, rd1, out_v,
          sem_s0, sem_s1, sem_d0, sem_d1):
        wid = lax.axis_index("s") * NC + lax.axis_index("c")
        wbase = wid * B_PER_W

        pltpu.sync_copy(src_hbm.at[pl.ds(wbase, B_PER_W)], idx_s)
        pltpu.sync_copy(dst_hbm.at[pl.ds(wbase, B_PER_W)], idx_d)

        bufs = ((rs0, rd0, sem_s0, sem_d0), (rs1, rd1, sem_s1, sem_d1))

        def issue(c, buf):
            rs, rd, ss, sd = buf
            cbase = pl.multiple_of(c * C, C)
            pltpu.async_copy(x_hbm.at[idx_s.at[pl.ds(cbase, C)]], rs, ss)
            pltpu.async_copy(x_hbm.at[idx_d.at[pl.ds(cbase, C)]], rd, sd)

        def wait_and_compute(c, buf):
            rs, rd, ss, sd = buf
            pltpu.make_async_copy(x_hbm.at[idx_s.at[pl.ds(0, C)]], rs, ss).wait()
            pltpu.make_async_copy(x_hbm.at[idx_d.at[pl.ds(0, C)]], rd, sd).wait()
            _compute_chunk(rs, rd, out_v, c * C)

        issue(0, bufs[0])

        def pair_body(i, _):
            c = i * 2
            issue(c + 1, bufs[1])
            wait_and_compute(c, bufs[0])
            issue(c + 2, bufs[0])
            wait_and_compute(c + 1, bufs[1])
            return 0

        lax.fori_loop(0, (N_CHUNKS - 1) // 2, pair_body, 0)
        wait_and_compute(N_CHUNKS - 1, bufs[0])

        pltpu.sync_copy(out_v, out_hbm.at[pl.ds(wbase, B_PER_W)])

    return k


_sc_kernel = _make_sc_kernel()


@jax.jit
def kernel(x, edge_index):
    ei = edge_index.astype(jnp.int32)
    xb = x.astype(jnp.bfloat16).reshape(N_NODES, D // 2, 2)
    xi = lax.bitcast_convert_type(xb, jnp.int32)
    return _sc_kernel(xi, ei[0], ei[1])


# speedup vs baseline: 1.3196x; 1.1604x over previous
import functools

import jax
import jax.numpy as jnp
from jax import lax
from jax.experimental import pallas as pl
from jax.experimental.pallas import tpu as pltpu
from jax.experimental.pallas import tpu_sc as plsc

N_NODES = 10000
D = 128
W = D // 2
B = 320000
NC = 2
NS = 16
NW = NC * NS
B_PER_W = B // NW
C = 80
N_CHUNKS = B_PER_W // C
G = C // 16
R_PER_T = N_NODES // NS
R_CHUNK = 125


def _convert_rows(x_hbm, tbl, tid, conv_in, conv_out):
    for cc in range(R_PER_T // R_CHUNK):
        rowbase = tid * R_PER_T + cc * R_CHUNK
        pltpu.sync_copy(x_hbm.at[pl.ds(rowbase, R_CHUNK), :], conv_in)

        def r_body(r, _):
            for k in range(4):
                a = conv_in[r, pl.ds(k * 16, 16)]
                b = conv_in[r, pl.ds(64 + k * 16, 16)]
                p = plsc.pack(a, b, format=plsc.PackFormat.INTERLEAVED)
                w = plsc.bitcast(p, jnp.int32)
                conv_out[r, pl.ds(k * 16, 16)] = w
            return 0

        lax.fori_loop(0, R_CHUNK, r_body, 0)
        pltpu.sync_copy(conv_out, tbl.at[pl.ds(rowbase, R_CHUNK), :])


def _compute_chunk(rows_s, rows_d, out_v, out_base):
    lane = lax.iota(jnp.int32, 16)

    def g_body(g, _):
        def q_body(q, tot):
            for jj in range(4):
                j = q * 4 + jj
                e = g * 16 + j
                acc0 = jnp.zeros((16,), jnp.float32)
                acc1 = jnp.zeros((16,), jnp.float32)
                for k in range(4):
                    s = plsc.bitcast(rows_s[e, pl.ds(k * 16, 16)], jnp.bfloat16)
                    t = plsc.bitcast(rows_d[e, pl.ds(k * 16, 16)], jnp.bfloat16)
                    a, b = plsc.unpack(s * t, format=plsc.PackFormat.INTERLEAVED)
                    acc0 = acc0 + a
                    acc1 = acc1 + b
                red = jnp.sum(acc0 + acc1)
                tot = jnp.where(lane == j, red, tot)
            return tot

        tot = lax.fori_loop(0, 4, q_body, jnp.zeros((16,), jnp.float32))
        out_v[pl.ds(out_base + g * 16, 16)] = tot
        return 0

    lax.fori_loop(0, G, g_body, 0)


def _make_sc_kernel():
    mesh = plsc.VectorSubcoreMesh(core_axis_name="c", subcore_axis_name="s")

    @functools.partial(
        pl.kernel,
        mesh=mesh,
        compiler_params=pltpu.CompilerParams(
            needs_layout_passes=False, use_tc_tiling_on_sc=False
        ),
        out_type=(
            jax.ShapeDtypeStruct((B,), jnp.float32),
            jax.ShapeDtypeStruct((N_NODES, W), jnp.int32),
            jax.ShapeDtypeStruct((N_NODES, W), jnp.int32),
        ),
        scratch_types=[
            pltpu.VMEM((R_CHUNK, D), jnp.float32),
            pltpu.VMEM((R_CHUNK, W), jnp.int32),
            pltpu.VMEM((B_PER_W,), jnp.int32),
            pltpu.VMEM((B_PER_W,), jnp.int32),
            pltpu.VMEM((C, W), jnp.int32),
            pltpu.VMEM((C, W), jnp.int32),
            pltpu.VMEM((C, W), jnp.int32),
            pltpu.VMEM((C, W), jnp.int32),
            pltpu.VMEM((B_PER_W,), jnp.float32),
            pltpu.SemaphoreType.DMA,
            pltpu.SemaphoreType.DMA,
            pltpu.SemaphoreType.DMA,
            pltpu.SemaphoreType.DMA,
        ],
    )
    def k(x_hbm, edge_hbm, out_hbm, tbl0, tbl1,
          conv_in, conv_out, idx_s, idx_d, rs0, rs1, rd0, rd1, out_v,
          sem_s0, sem_s1, sem_d0, sem_d1):
        cid = lax.axis_index("c")
        sid = lax.axis_index("s")
        wid = sid * NC + cid
        wbase = wid * B_PER_W

        @pl.when(cid == 0)
        def _():
            _convert_rows(x_hbm, tbl0, sid, conv_in, conv_out)

        @pl.when(cid == 1)
        def _():
            _convert_rows(x_hbm, tbl1, sid, conv_in, conv_out)

        pltpu.sync_copy(edge_hbm.at[0, pl.ds(wbase, B_PER_W)], idx_s)
        pltpu.sync_copy(edge_hbm.at[1, pl.ds(wbase, B_PER_W)], idx_d)
        plsc.subcore_barrier()

        bufs = ((rs0, rd0, sem_s0, sem_d0), (rs1, rd1, sem_s1, sem_d1))

        def issue(c, buf):
            rs, rd, ss, sd = buf
            cbase = pl.multiple_of(c * C, C)

            @pl.when(cid == 0)
            def _():
                pltpu.async_copy(tbl0.at[idx_s.at[pl.ds(cbase, C)]], rs, ss)
                pltpu.async_copy(tbl0.at[idx_d.at[pl.ds(cbase, C)]], rd, sd)

            @pl.when(cid == 1)
            def _():
                pltpu.async_copy(tbl1.at[idx_s.at[pl.ds(cbase, C)]], rs, ss)
                pltpu.async_copy(tbl1.at[idx_d.at[pl.ds(cbase, C)]], rd, sd)

        def wait_and_compute(c, buf):
            rs, rd, ss, sd = buf
            pltpu.make_async_copy(tbl0.at[idx_s.at[pl.ds(0, C)]], rs, ss).wait()
            pltpu.make_async_copy(tbl0.at[idx_d.at[pl.ds(0, C)]], rd, sd).wait()
            _compute_chunk(rs, rd, out_v, c * C)

        issue(0, bufs[0])

        def pair_body(i, _):
            c = i * 2
            issue(c + 1, bufs[1])
            wait_and_compute(c, bufs[0])
            issue(c + 2, bufs[0])
            wait_and_compute(c + 1, bufs[1])
            return 0

        lax.fori_loop(0, (N_CHUNKS - 1) // 2, pair_body, 0)
        wait_and_compute(N_CHUNKS - 1, bufs[0])

        pltpu.sync_copy(out_v, out_hbm.at[pl.ds(wbase, B_PER_W)])

    return k


_sc_kernel = _make_sc_kernel()


@jax.jit
def kernel(x, edge_index):
    scores, _, _ = _sc_kernel(x, edge_index.astype(jnp.int32))
    return scores


# speedup vs baseline: 1.4050x; 1.0647x over previous
import functools

import jax
import jax.numpy as jnp
from jax import lax
from jax.experimental import pallas as pl
from jax.experimental.pallas import tpu as pltpu
from jax.experimental.pallas import tpu_sc as plsc

N_NODES = 10000
D = 128
W = D // 2
B = 320000
NC = 2
NS = 16
NW = NC * NS
B_PER_W = B // NW
C = 80
N_CHUNKS = B_PER_W // C
G = C // 16
R_PER_T = N_NODES // NS
R_CHUNK = 125


def _convert_rows(x_hbm, tbl, tid, cins, couts, lsems, ssems):
    n_cc = R_PER_T // R_CHUNK

    def load(cc, p):
        base = tid * R_PER_T + cc * R_CHUNK
        pltpu.async_copy(x_hbm.at[pl.ds(base, R_CHUNK), :], cins[p], lsems[p])

    load(0, 0)
    for cc in range(n_cc):
        p = cc % 2
        if cc + 1 < n_cc:
            load(cc + 1, 1 - p)
        pltpu.make_async_copy(
            x_hbm.at[pl.ds(0, R_CHUNK), :], cins[p], lsems[p]
        ).wait()
        if cc >= 2:
            pltpu.make_async_copy(
                couts[p], tbl.at[pl.ds(0, R_CHUNK), :], ssems[p]
            ).wait()
        conv_in, conv_out = cins[p], couts[p]

        def r_body(r, _):
            for k in range(4):
                a = conv_in[r, pl.ds(k * 16, 16)]
                b = conv_in[r, pl.ds(64 + k * 16, 16)]
                pk = plsc.pack(a, b, format=plsc.PackFormat.INTERLEAVED)
                conv_out[r, pl.ds(k * 16, 16)] = plsc.bitcast(pk, jnp.int32)
            return 0

        lax.fori_loop(0, R_CHUNK, r_body, 0)
        base = tid * R_PER_T + cc * R_CHUNK
        pltpu.async_copy(couts[p], tbl.at[pl.ds(base, R_CHUNK), :], ssems[p])
    for p in range(2):
        pltpu.make_async_copy(
            couts[p], tbl.at[pl.ds(0, R_CHUNK), :], ssems[p]
        ).wait()


def _compute_chunk(rows_s, rows_d, out_v, out_base):
    lane = lax.iota(jnp.int32, 16)

    def g_body(g, _):
        def q_body(q, tot):
            for jj in range(4):
                j = q * 4 + jj
                e = g * 16 + j
                acc0 = jnp.zeros((16,), jnp.float32)
                acc1 = jnp.zeros((16,), jnp.float32)
                for k in range(4):
                    s = plsc.bitcast(rows_s[e, pl.ds(k * 16, 16)], jnp.bfloat16)
                    t = plsc.bitcast(rows_d[e, pl.ds(k * 16, 16)], jnp.bfloat16)
                    a, b = plsc.unpack(s * t, format=plsc.PackFormat.INTERLEAVED)
                    acc0 = acc0 + a
                    acc1 = acc1 + b
                red = jnp.sum(acc0 + acc1)
                tot = jnp.where(lane == j, red, tot)
            return tot

        tot = lax.fori_loop(0, 4, q_body, jnp.zeros((16,), jnp.float32))
        out_v[pl.ds(out_base + g * 16, 16)] = tot
        return 0

    lax.fori_loop(0, G, g_body, 0)


def _make_sc_kernel():
    mesh = plsc.VectorSubcoreMesh(core_axis_name="c", subcore_axis_name="s")

    @functools.partial(
        pl.kernel,
        mesh=mesh,
        compiler_params=pltpu.CompilerParams(
            needs_layout_passes=False, use_tc_tiling_on_sc=False
        ),
        out_type=(
            jax.ShapeDtypeStruct((B,), jnp.float32),
            jax.ShapeDtypeStruct((N_NODES, W), jnp.int32),
            jax.ShapeDtypeStruct((N_NODES, W), jnp.int32),
        ),
        scratch_types=[
            pltpu.VMEM((R_CHUNK, D), jnp.float32),
            pltpu.VMEM((R_CHUNK, D), jnp.float32),
            pltpu.VMEM((R_CHUNK, W), jnp.int32),
            pltpu.VMEM((R_CHUNK, W), jnp.int32),
            pltpu.VMEM((B_PER_W,), jnp.int32),
            pltpu.VMEM((B_PER_W,), jnp.int32),
            pltpu.VMEM((C, W), jnp.int32),
            pltpu.VMEM((C, W), jnp.int32),
            pltpu.VMEM((C, W), jnp.int32),
            pltpu.VMEM((C, W), jnp.int32),
            pltpu.VMEM((B_PER_W,), jnp.float32),
            pltpu.SemaphoreType.DMA,
            pltpu.SemaphoreType.DMA,
            pltpu.SemaphoreType.DMA,
            pltpu.SemaphoreType.DMA,
            pltpu.SemaphoreType.DMA,
            pltpu.SemaphoreType.DMA,
            pltpu.SemaphoreType.DMA,
            pltpu.SemaphoreType.DMA,
            pltpu.SemaphoreType.DMA,
            pltpu.SemaphoreType.DMA,
        ],
    )
    def k(x_hbm, edge_hbm, out_hbm, tbl0, tbl1,
          cin0, cin1, cout0, cout1, idx_s, idx_d, rs0, rs1, rd0, rd1, out_v,
          sem_s0, sem_s1, sem_d0, sem_d1,
          cl0, cl1, cs0, cs1, sem_is, sem_id):
        cid = lax.axis_index("c")
        sid = lax.axis_index("s")
        wid = sid * NC + cid
        wbase = wid * B_PER_W

        pltpu.async_copy(edge_hbm.at[0, pl.ds(wbase, B_PER_W)], idx_s, sem_is)
        pltpu.async_copy(edge_hbm.at[1, pl.ds(wbase, B_PER_W)], idx_d, sem_id)

        @pl.when(cid == 0)
        def _():
            _convert_rows(x_hbm, tbl0, sid, (cin0, cin1), (cout0, cout1),
                          (cl0, cl1), (cs0, cs1))

        @pl.when(cid == 1)
        def _():
            _convert_rows(x_hbm, tbl1, sid, (cin0, cin1), (cout0, cout1),
                          (cl0, cl1), (cs0, cs1))

        pltpu.make_async_copy(
            edge_hbm.at[0, pl.ds(0, B_PER_W)], idx_s, sem_is
        ).wait()
        pltpu.make_async_copy(
            edge_hbm.at[1, pl.ds(0, B_PER_W)], idx_d, sem_id
        ).wait()
        plsc.subcore_barrier()

        bufs = ((rs0, rd0, sem_s0, sem_d0), (rs1, rd1, sem_s1, sem_d1))

        def issue(c, buf):
            rs, rd, ss, sd = buf
            cbase = pl.multiple_of(c * C, C)

            @pl.when(cid == 0)
            def _():
                pltpu.async_copy(tbl0.at[idx_s.at[pl.ds(cbase, C)]], rs, ss)
                pltpu.async_copy(tbl0.at[idx_d.at[pl.ds(cbase, C)]], rd, sd)

            @pl.when(cid == 1)
            def _():
                pltpu.async_copy(tbl1.at[idx_s.at[pl.ds(cbase, C)]], rs, ss)
                pltpu.async_copy(tbl1.at[idx_d.at[pl.ds(cbase, C)]], rd, sd)

        def wait_and_compute(c, buf):
            rs, rd, ss, sd = buf
            pltpu.make_async_copy(tbl0.at[idx_s.at[pl.ds(0, C)]], rs, ss).wait()
            pltpu.make_async_copy(tbl0.at[idx_d.at[pl.ds(0, C)]], rd, sd).wait()
            _compute_chunk(rs, rd, out_v, c * C)

        issue(0, bufs[0])

        def pair_body(i, _):
            c = i * 2
            issue(c + 1, bufs[1])
            wait_and_compute(c, bufs[0])
            issue(c + 2, bufs[0])
            wait_and_compute(c + 1, bufs[1])
            return 0

        lax.fori_loop(0, (N_CHUNKS - 1) // 2, pair_body, 0)
        wait_and_compute(N_CHUNKS - 1, bufs[0])

        pltpu.sync_copy(out_v, out_hbm.at[pl.ds(wbase, B_PER_W)])

    return k


_sc_kernel = _make_sc_kernel()


@jax.jit
def kernel(x, edge_index):
    scores, _, _ = _sc_kernel(x, edge_index.astype(jnp.int32))
    return scores


# speedup vs baseline: 1.6227x; 1.1550x over previous
import functools

import jax
import jax.numpy as jnp
from jax import lax
from jax.experimental import pallas as pl
from jax.experimental.pallas import tpu as pltpu
from jax.experimental.pallas import tpu_sc as plsc

N_NODES = 10000
D = 128
W = D // 2
B = 320000
NC = 2
NS = 16
NW = NC * NS
B_PER_W = B // NW
C = 320
NFULL = B_PER_W // C
CT = B_PER_W - NFULL * C
R_PER_T = N_NODES // NS
R_CHUNK = 25


def _convert_rows(x_hbm, tbl, tid, cins, couts, lsems, ssems):
    n_cc = R_PER_T // R_CHUNK

    def load(cc, p):
        base = tid * R_PER_T + cc * R_CHUNK
        pltpu.async_copy(x_hbm.at[pl.ds(base, R_CHUNK), :], cins[p], lsems[p])

    load(0, 0)

    def cc_body(cc2, _):
        for q in range(2):
            cc = cc2 * 2 + q
            load(cc + 1, 1 - q)
            pltpu.make_async_copy(
                x_hbm.at[pl.ds(0, R_CHUNK), :], cins[q], lsems[q]
            ).wait()

            @pl.when(cc >= 2)
            def _():
                pltpu.make_async_copy(
                    couts[q], tbl.at[pl.ds(0, R_CHUNK), :], ssems[q]
                ).wait()

            conv_in, conv_out = cins[q], couts[q]

            def r_body(r, _):
                for k in range(4):
                    a = conv_in[r, pl.ds(k * 16, 16)]
                    b = conv_in[r, pl.ds(64 + k * 16, 16)]
                    pk = plsc.pack(a, b, format=plsc.PackFormat.INTERLEAVED)
                    conv_out[r, pl.ds(k * 16, 16)] = plsc.bitcast(pk, jnp.int32)
                return 0

            lax.fori_loop(0, R_CHUNK, r_body, 0)
            base = tid * R_PER_T + cc * R_CHUNK
            pltpu.async_copy(couts[q], tbl.at[pl.ds(base, R_CHUNK), :], ssems[q])
        return 0

    lax.fori_loop(0, (n_cc - 1) // 2, cc_body, 0)
    cc = n_cc - 1
    pltpu.make_async_copy(
        x_hbm.at[pl.ds(0, R_CHUNK), :], cins[0], lsems[0]
    ).wait()
    pltpu.make_async_copy(
        couts[0], tbl.at[pl.ds(0, R_CHUNK), :], ssems[0]
    ).wait()
    conv_in, conv_out = cins[0], couts[0]

    def r_tail(r, _):
        for k in range(4):
            a = conv_in[r, pl.ds(k * 16, 16)]
            b = conv_in[r, pl.ds(64 + k * 16, 16)]
            pk = plsc.pack(a, b, format=plsc.PackFormat.INTERLEAVED)
            conv_out[r, pl.ds(k * 16, 16)] = plsc.bitcast(pk, jnp.int32)
        return 0

    lax.fori_loop(0, R_CHUNK, r_tail, 0)
    base = tid * R_PER_T + cc * R_CHUNK
    pltpu.async_copy(couts[0], tbl.at[pl.ds(base, R_CHUNK), :], ssems[0])
    for q in range(2):
        pltpu.make_async_copy(
            couts[q], tbl.at[pl.ds(0, R_CHUNK), :], ssems[q]
        ).wait()


def _compute_chunk(rows_s, rows_d, out_v, out_base, n_groups):
    lane = lax.iota(jnp.int32, 16)

    def g_body(g, _):
        def q_body(q, tot):
            for jj in range(4):
                j = q * 4 + jj
                e = g * 16 + j
                acc0 = jnp.zeros((16,), jnp.float32)
                acc1 = jnp.zeros((16,), jnp.float32)
                for k in range(4):
                    s = plsc.bitcast(rows_s[e, pl.ds(k * 16, 16)], jnp.bfloat16)
                    t = plsc.bitcast(rows_d[e, pl.ds(k * 16, 16)], jnp.bfloat16)
                    a, b = plsc.unpack(s * t, format=plsc.PackFormat.INTERLEAVED)
                    acc0 = acc0 + a
                    acc1 = acc1 + b
                red = jnp.sum(acc0 + acc1)
                tot = jnp.where(lane == j, red, tot)
            return tot

        tot = lax.fori_loop(0, 4, q_body, jnp.zeros((16,), jnp.float32))
        out_v[pl.ds(out_base + g * 16, 16)] = tot
        return 0

    lax.fori_loop(0, n_groups, g_body, 0)


def _make_sc_kernel():
    mesh = plsc.VectorSubcoreMesh(core_axis_name="c", subcore_axis_name="s")

    @functools.partial(
        pl.kernel,
        mesh=mesh,
        compiler_params=pltpu.CompilerParams(
            needs_layout_passes=False, use_tc_tiling_on_sc=False
        ),
        out_type=(
            jax.ShapeDtypeStruct((B,), jnp.float32),
            jax.ShapeDtypeStruct((N_NODES, W), jnp.int32),
            jax.ShapeDtypeStruct((N_NODES, W), jnp.int32),
        ),
        scratch_types=[
            pltpu.VMEM((R_CHUNK, D), jnp.float32),
            pltpu.VMEM((R_CHUNK, D), jnp.float32),
            pltpu.VMEM((R_CHUNK, W), jnp.int32),
            pltpu.VMEM((R_CHUNK, W), jnp.int32),
            pltpu.VMEM((B_PER_W,), jnp.int32),
            pltpu.VMEM((B_PER_W,), jnp.int32),
            pltpu.VMEM((C, W), jnp.int32),
            pltpu.VMEM((C, W), jnp.int32),
            pltpu.VMEM((C, W), jnp.int32),
            pltpu.VMEM((C, W), jnp.int32),
            pltpu.VMEM((B_PER_W,), jnp.float32),
            pltpu.SemaphoreType.DMA,
            pltpu.SemaphoreType.DMA,
            pltpu.SemaphoreType.DMA,
            pltpu.SemaphoreType.DMA,
            pltpu.SemaphoreType.DMA,
            pltpu.SemaphoreType.DMA,
            pltpu.SemaphoreType.DMA,
            pltpu.SemaphoreType.DMA,
            pltpu.SemaphoreType.DMA,
            pltpu.SemaphoreType.DMA,
        ],
    )
    def k(x_hbm, edge_hbm, out_hbm, tbl0, tbl1,
          cin0, cin1, cout0, cout1, idx_s, idx_d, rs0, rs1, rd0, rd1, out_v,
          sem_s0, sem_s1, sem_d0, sem_d1,
          cl0, cl1, cs0, cs1, sem_is, sem_id):
        cid = lax.axis_index("c")
        sid = lax.axis_index("s")
        wid = sid * NC + cid
        wbase = wid * B_PER_W

        pltpu.async_copy(edge_hbm.at[pl.ds(wbase, B_PER_W)], idx_s, sem_is)
        pltpu.async_copy(edge_hbm.at[pl.ds(B + wbase, B_PER_W)], idx_d, sem_id)

        @pl.when(cid == 0)
        def _():
            _convert_rows(x_hbm, tbl0, sid, (cin0, cin1), (cout0, cout1),
                          (cl0, cl1), (cs0, cs1))

        @pl.when(cid == 1)
        def _():
            _convert_rows(x_hbm, tbl1, sid, (cin0, cin1), (cout0, cout1),
                          (cl0, cl1), (cs0, cs1))

        pltpu.make_async_copy(
            edge_hbm.at[pl.ds(0, B_PER_W)], idx_s, sem_is
        ).wait()
        pltpu.make_async_copy(
            edge_hbm.at[pl.ds(0, B_PER_W)], idx_d, sem_id
        ).wait()
        plsc.subcore_barrier()

        bufs = ((rs0, rd0, sem_s0, sem_d0), (rs1, rd1, sem_s1, sem_d1))

        def issue(c, buf, n):
            rs, rd, ss, sd = buf
            cbase = pl.multiple_of(c * C, 16)
            rs_d = rs.at[pl.ds(0, n), :]
            rd_d = rd.at[pl.ds(0, n), :]

            @pl.when(cid == 0)
            def _():
                pltpu.async_copy(tbl0.at[idx_s.at[pl.ds(cbase, n)]], rs_d, ss)
                pltpu.async_copy(tbl0.at[idx_d.at[pl.ds(cbase, n)]], rd_d, sd)

            @pl.when(cid == 1)
            def _():
                pltpu.async_copy(tbl1.at[idx_s.at[pl.ds(cbase, n)]], rs_d, ss)
                pltpu.async_copy(tbl1.at[idx_d.at[pl.ds(cbase, n)]], rd_d, sd)

        def wait_and_compute(c, buf, n):
            rs, rd, ss, sd = buf
            pltpu.make_async_copy(
                tbl0.at[idx_s.at[pl.ds(0, n)]], rs.at[pl.ds(0, n), :], ss
            ).wait()
            pltpu.make_async_copy(
                tbl0.at[idx_d.at[pl.ds(0, n)]], rd.at[pl.ds(0, n), :], sd
            ).wait()
            _compute_chunk(rs, rd, out_v, c * C, n // 16)

        issue(0, bufs[0], C)

        def pair_body(i, _):
            c = i * 2
            issue(c + 1, bufs[1], C)
            wait_and_compute(c, bufs[0], C)
            issue(c + 2, bufs[0], C)
            wait_and_compute(c + 1, bufs[1], C)
            return 0

        lax.fori_loop(0, (NFULL - 1) // 2, pair_body, 0)
        issue(NFULL, bufs[1], CT)
        wait_and_compute(NFULL - 1, bufs[0], C)
        wait_and_compute(NFULL, bufs[1], CT)

        pltpu.sync_copy(out_v, out_hbm.at[pl.ds(wbase, B_PER_W)])

    return k


_sc_kernel = _make_sc_kernel()


@jax.jit
def kernel(x, edge_index):
    ei = edge_index.astype(jnp.int32).reshape(-1)
    scores, _, _ = _sc_kernel(x, ei)
    return scores
